# baseline (speedup 1.0000x reference)
import functools

import jax
import jax.numpy as jnp
from jax import lax
from jax.experimental import pallas as pl
from jax.experimental.pallas import tpu as pltpu

_C = 5
_LANES = 128


def _gumbel_argmax(logits, u):
    score = logits - jnp.log(-jnp.log(u))
    mx = jnp.max(score)
    ii = lax.broadcasted_iota(jnp.int32, (1, _LANES), 1)
    return jnp.min(jnp.where(score == mx, ii, _LANES))


def _body(pm_ref, ps_ref, um_ref, us_ref, x_ref, o_ref):
    mu_idx = _gumbel_argmax(pm_ref[...], um_ref[...])
    sig_idx = _gumbel_argmax(ps_ref[...], us_ref[...])
    a = mu_idx.astype(jnp.float32)
    b = jnp.exp(sig_idx.astype(jnp.float32))
    o_ref[...] = a + b * x_ref[...]


def _manual_body(d, w2, steps):
    def body(pm_ref, ps_ref, um_ref, us_ref, x_hbm, o_hbm,
             inb, outb, insem, outsem):
        mu_idx = _gumbel_argmax(pm_ref[...], um_ref[...])
        sig_idx = _gumbel_argmax(ps_ref[...], us_ref[...])
        a = mu_idx.astype(jnp.float32)
        b = jnp.exp(sig_idx.astype(jnp.float32))

        def in_copy(off, s):
            return pltpu.make_async_copy(
                x_hbm.at[:, pl.ds(off, w2)], inb.at[s], insem.at[s])

        def out_copy(off, s):
            return pltpu.make_async_copy(
                outb.at[s], o_hbm.at[:, pl.ds(off, w2)], outsem.at[s])

        for s in range(d):
            in_copy(s * w2, s).start()

        def loop(g2, carry):
            for s in range(d):
                g = g2 * d + s
                off = pl.multiple_of(g * w2, w2)
                in_copy(off, s).wait()

                @pl.when(g2 > 0)
                def _wait_prev_out():
                    out_copy(0, s).wait()

                outb[s] = a + b * inb[s]
                out_copy(off, s).start()

                @pl.when(g + d < steps)
                def _start_next_in():
                    nxt = pl.multiple_of((g + d) * w2, w2)
                    in_copy(nxt, s).start()
            return carry

        lax.fori_loop(0, steps // d, loop, 0)
        for s in range(d):
            out_copy((steps - d + s) * w2, s).wait()

    return body


@functools.partial(jax.jit, static_argnames=("d", "w2"))
def _run_manual(pm, ps, um, us, x_t, d, w2):
    n = x_t.shape[1]
    steps = n // w2
    return pl.pallas_call(
        _manual_body(d, w2, steps),
        in_specs=[
            pl.BlockSpec(memory_space=pltpu.MemorySpace.VMEM),
            pl.BlockSpec(memory_space=pltpu.MemorySpace.VMEM),
            pl.BlockSpec(memory_space=pltpu.MemorySpace.VMEM),
            pl.BlockSpec(memory_space=pltpu.MemorySpace.VMEM),
            pl.BlockSpec(memory_space=pltpu.MemorySpace.HBM),
        ],
        out_specs=pl.BlockSpec(memory_space=pltpu.MemorySpace.HBM),
        out_shape=jax.ShapeDtypeStruct((_C, n), jnp.float32),
        scratch_shapes=[
            pltpu.VMEM((d, _C, w2), jnp.float32),
            pltpu.VMEM((d, _C, w2), jnp.float32),
            pltpu.SemaphoreType.DMA((d,)),
            pltpu.SemaphoreType.DMA((d,)),
        ],
    )(pm, ps, um, us, x_t)


@functools.partial(jax.jit, static_argnames=("w",))
def _run(pm, ps, um, us, x_t, w):
    n = x_t.shape[1]
    grid = (n // w,)
    param_spec = pl.BlockSpec((1, _LANES), lambda i: (0, 0))
    return pl.pallas_call(
        _body,
        grid=grid,
        in_specs=[
            param_spec, param_spec, param_spec, param_spec,
            pl.BlockSpec((_C, w), lambda i: (0, i)),
        ],
        out_specs=pl.BlockSpec((_C, w), lambda i: (0, i)),
        out_shape=jax.ShapeDtypeStruct((_C, n), jnp.float32),
        compiler_params=pltpu.CompilerParams(
            dimension_semantics=("arbitrary",),
        ),
    )(pm, ps, um, us, x_t)


def kernel(prob_mu, prob_sigma, u_mu, u_sigma, eps):
    n = eps.shape[0]

    def pad128(v, fill):
        return jnp.concatenate(
            [v, jnp.full((_LANES - _C,), fill, v.dtype)]).reshape(1, _LANES)

    pm = pad128(prob_mu, -1e30)
    ps = pad128(prob_sigma, -1e30)
    um = pad128(u_mu, 0.5)
    us = pad128(u_sigma, 0.5)

    out_t = _run_manual(pm, ps, um, us, eps.T, d=8, w2=65536)
    return out_t.T


# speedup vs baseline: 1.0024x; 1.0024x over previous
import functools

import jax
import jax.numpy as jnp
from jax import lax
from jax.experimental import pallas as pl
from jax.experimental.pallas import tpu as pltpu

_C = 5
_LANES = 128


def _gumbel_argmax(logits, u):
    score = logits - jnp.log(-jnp.log(u))
    mx = jnp.max(score)
    ii = lax.broadcasted_iota(jnp.int32, (1, _LANES), 1)
    return jnp.min(jnp.where(score == mx, ii, _LANES))


def _pipeline_body(d, w2, steps):
    def body(pm_ref, ps_ref, um_ref, us_ref, x_hbm, o_hbm,
             inb, outb, insem, outsem):
        mu_idx = _gumbel_argmax(pm_ref[...], um_ref[...])
        sig_idx = _gumbel_argmax(ps_ref[...], us_ref[...])
        a = mu_idx.astype(jnp.float32)
        b = jnp.exp(sig_idx.astype(jnp.float32))

        def in_copy(off, s):
            return pltpu.make_async_copy(
                x_hbm.at[:, pl.ds(off, w2)], inb.at[s], insem.at[s])

        def out_copy(off, s):
            return pltpu.make_async_copy(
                outb.at[s], o_hbm.at[:, pl.ds(off, w2)], outsem.at[s])

        for s in range(d):
            in_copy(s * w2, s).start()

        def loop(g2, carry):
            for s in range(d):
                g = g2 * d + s
                off = pl.multiple_of(g * w2, w2)
                in_copy(off, s).wait()

                @pl.when(g2 > 0)
                def _wait_prev_out():
                    out_copy(0, s).wait()

                outb[s] = a + b * inb[s]
                out_copy(off, s).start()

                @pl.when(g + d < steps)
                def _start_next_in():
                    nxt = pl.multiple_of((g + d) * w2, w2)
                    in_copy(nxt, s).start()
            return carry

        lax.fori_loop(0, steps // d, loop, 0)
        for s in range(d):
            out_copy((steps - d + s) * w2, s).wait()

    return body


@functools.partial(jax.jit, static_argnames=("d", "w2"))
def _run_pipelined(pm, ps, um, us, x_t, d, w2):
    n = x_t.shape[1]
    steps = n // w2
    return pl.pallas_call(
        _pipeline_body(d, w2, steps),
        in_specs=[
            pl.BlockSpec(memory_space=pltpu.MemorySpace.VMEM),
            pl.BlockSpec(memory_space=pltpu.MemorySpace.VMEM),
            pl.BlockSpec(memory_space=pltpu.MemorySpace.VMEM),
            pl.BlockSpec(memory_space=pltpu.MemorySpace.VMEM),
            pl.BlockSpec(memory_space=pltpu.MemorySpace.HBM),
        ],
        out_specs=pl.BlockSpec(memory_space=pltpu.MemorySpace.HBM),
        out_shape=jax.ShapeDtypeStruct((_C, n), jnp.float32),
        scratch_shapes=[
            pltpu.VMEM((d, _C, w2), jnp.float32),
            pltpu.VMEM((d, _C, w2), jnp.float32),
            pltpu.SemaphoreType.DMA((d,)),
            pltpu.SemaphoreType.DMA((d,)),
        ],
    )(pm, ps, um, us, x_t)


def kernel(prob_mu, prob_sigma, u_mu, u_sigma, eps):
    def pad128(v, fill):
        return jnp.concatenate(
            [v, jnp.full((_LANES - _C,), fill, v.dtype)]).reshape(1, _LANES)

    pm = pad128(prob_mu, -1e30)
    ps = pad128(prob_sigma, -1e30)
    um = pad128(u_mu, 0.5)
    us = pad128(u_sigma, 0.5)

    out_t = _run_pipelined(pm, ps, um, us, eps.T, d=4, w2=131072)
    return out_t.T
